# Initial kernel scaffold; baseline (speedup 1.0000x reference)
import jax
import jax.numpy as jnp
from jax.experimental import pallas as pl


def kernel(x, base_w0, spline_w0, spline_s0, base_w1, spline_w1, spline_s1, ln_g, ln_b):
    raise NotImplementedError("write your pallas kernel here")


# speedup vs baseline: 6.1909x; 6.1909x over previous
import jax
import jax.numpy as jnp
import numpy as np
from jax.experimental import pallas as pl
from jax.experimental.pallas import tpu as pltpu

_GRID_SIZE = 5
_ORDER = 3
_GK = _GRID_SIZE + _ORDER
_NKNOT = _GRID_SIZE + 2 * _ORDER + 1
_H = np.float32(2.0 / _GRID_SIZE)
_T0 = np.float32(-_ORDER) * _H + np.float32(-1.0)
_INV_H = np.float32(1.0) / _H
_LN_EPS = np.float32(1e-5)

_TB = 256


def _bases_concat(x):
    y = (x - _T0) * _INV_H
    d = [y - np.float32(j) for j in range(_NKNOT)]
    e = [np.float32(j) - y for j in range(_NKNOT)]
    zero = jnp.zeros_like(y)
    b = [jnp.maximum(jnp.minimum(d[i], e[i + 2]), zero) for i in range(10)]
    for k in (2, 3):
        b = [d[i] * b[i] - d[i + k + 1] * b[i + 1] for i in range(len(b) - 1)]
    return jnp.concatenate(b, axis=1)


def _body(x_ref, bw0_ref, sw0_ref, bw1_ref, sw1_ref, g_ref, b_ref, o_ref):
    f32 = jnp.float32
    x = x_ref[...]
    h1 = jnp.dot(jax.nn.gelu(x), bw0_ref[...], preferred_element_type=f32)
    h1 = h1 + jnp.dot(_bases_concat(x), sw0_ref[...], preferred_element_type=f32)
    h2 = jnp.dot(jax.nn.gelu(h1), bw1_ref[...], preferred_element_type=f32)
    h2 = h2 + jnp.dot(_bases_concat(h1), sw1_ref[...], preferred_element_type=f32)
    mu = jnp.mean(h2, axis=-1, keepdims=True)
    xc = h2 - mu
    var = jnp.mean(xc * xc, axis=-1, keepdims=True)
    o_ref[...] = xc * jax.lax.rsqrt(var + _LN_EPS) * g_ref[...] + b_ref[...]


def kernel(x, base_w0, spline_w0, spline_s0, base_w1, spline_w1, spline_s1,
           ln_g, ln_b):
    B, D0 = x.shape
    D1 = base_w0.shape[0]
    D2 = base_w1.shape[0]

    bw0t = base_w0.T
    bw1t = base_w1.T
    sw0 = spline_w0 * (spline_s0 * np.float32(1.0 / 6.0))[..., None]
    sw0 = sw0.transpose(2, 1, 0).reshape(_GK * D0, D1)
    sw1 = spline_w1 * (spline_s1 * np.float32(1.0 / 6.0))[..., None]
    sw1 = sw1.transpose(2, 1, 0).reshape(_GK * D1, D2)
    g2 = ln_g.reshape(1, D2)
    b2 = ln_b.reshape(1, D2)

    grid = (B // _TB,)
    full = lambda i: (0, 0)
    out = pl.pallas_call(
        _body,
        grid=grid,
        in_specs=[
            pl.BlockSpec((_TB, D0), lambda i: (i, 0)),
            pl.BlockSpec((D0, D1), full),
            pl.BlockSpec((_GK * D0, D1), full),
            pl.BlockSpec((D1, D2), full),
            pl.BlockSpec((_GK * D1, D2), full),
            pl.BlockSpec((1, D2), full),
            pl.BlockSpec((1, D2), full),
        ],
        out_specs=pl.BlockSpec((_TB, D2), lambda i: (i, 0)),
        out_shape=jax.ShapeDtypeStruct((B, D2), jnp.float32),
        compiler_params=pltpu.CompilerParams(
            dimension_semantics=("parallel",),
        ),
    )(x, bw0t, sw0, bw1t, sw1, g2, b2)
    return out


# speedup vs baseline: 6.3060x; 1.0186x over previous
import jax
import jax.numpy as jnp
import numpy as np
from jax.experimental import pallas as pl
from jax.experimental.pallas import tpu as pltpu

_GRID_SIZE = 5
_ORDER = 3
_GK = _GRID_SIZE + _ORDER
_NKNOT = _GRID_SIZE + 2 * _ORDER + 1
_H = np.float32(2.0 / _GRID_SIZE)
_T0 = np.float32(-_ORDER) * _H + np.float32(-1.0)
_INV_H = np.float32(1.0) / _H
_LN_EPS = np.float32(1e-5)

_TB = 256


def _bases_concat(x):
    y = (x - _T0) * _INV_H
    d = [y - np.float32(j) for j in range(_NKNOT)]
    e = [np.float32(j) - y for j in range(_NKNOT)]
    zero = jnp.zeros_like(y)
    b = [jnp.maximum(jnp.minimum(d[i], e[i + 2]), zero) for i in range(10)]
    for k in (2, 3):
        b = [d[i] * b[i] - d[i + k + 1] * b[i + 1] for i in range(len(b) - 1)]
    return jnp.concatenate(b, axis=1)


def _mdot(a, b):
    return jax.lax.dot_general(a, b, (((1,), (0,)), ((), ())),
                               preferred_element_type=jnp.float32)


def _body(x_ref, bw0_ref, sw0_ref, bw1_ref, sw1_ref, g_ref, b_ref, o_ref):
    x = x_ref[...]
    h1 = _mdot(jax.nn.gelu(x), bw0_ref[...])
    h1 = h1 + _mdot(_bases_concat(x), sw0_ref[...])
    h2 = _mdot(jax.nn.gelu(h1), bw1_ref[...])
    h2 = h2 + _mdot(_bases_concat(h1), sw1_ref[...])
    mu = jnp.mean(h2, axis=-1, keepdims=True)
    xc = h2 - mu
    var = jnp.mean(xc * xc, axis=-1, keepdims=True)
    o_ref[...] = xc * jax.lax.rsqrt(var + _LN_EPS) * g_ref[...] + b_ref[...]


def kernel(x, base_w0, spline_w0, spline_s0, base_w1, spline_w1, spline_s1,
           ln_g, ln_b):
    B, D0 = x.shape
    D1 = base_w0.shape[0]
    D2 = base_w1.shape[0]

    bf16 = jnp.bfloat16
    bw0t = base_w0.T.astype(bf16)
    bw1t = base_w1.T.astype(bf16)
    sw0 = spline_w0 * (spline_s0 * np.float32(1.0 / 6.0))[..., None]
    sw0 = sw0.transpose(2, 1, 0).reshape(_GK * D0, D1).astype(bf16)
    sw1 = spline_w1 * (spline_s1 * np.float32(1.0 / 6.0))[..., None]
    sw1 = sw1.transpose(2, 1, 0).reshape(_GK * D1, D2).astype(bf16)
    g2 = ln_g.reshape(1, D2)
    b2 = ln_b.reshape(1, D2)

    grid = (B // _TB,)
    full = lambda i: (0, 0)
    out = pl.pallas_call(
        _body,
        grid=grid,
        in_specs=[
            pl.BlockSpec((_TB, D0), lambda i: (i, 0)),
            pl.BlockSpec((D0, D1), full),
            pl.BlockSpec((_GK * D0, D1), full),
            pl.BlockSpec((D1, D2), full),
            pl.BlockSpec((_GK * D1, D2), full),
            pl.BlockSpec((1, D2), full),
            pl.BlockSpec((1, D2), full),
        ],
        out_specs=pl.BlockSpec((_TB, D2), lambda i: (i, 0)),
        out_shape=jax.ShapeDtypeStruct((B, D2), jnp.float32),
        compiler_params=pltpu.CompilerParams(
            dimension_semantics=("parallel",),
        ),
    )(x, bw0t, sw0, bw1t, sw1, g2, b2)
    return out


# speedup vs baseline: 6.6701x; 1.0577x over previous
import jax
import jax.numpy as jnp
import numpy as np
from jax.experimental import pallas as pl
from jax.experimental.pallas import tpu as pltpu

_GRID_SIZE = 5
_ORDER = 3
_GK = _GRID_SIZE + _ORDER
_NKNOT = _GRID_SIZE + 2 * _ORDER + 1
_H = np.float32(2.0 / _GRID_SIZE)
_T0 = np.float32(-_ORDER) * _H + np.float32(-1.0)
_INV_H = np.float32(1.0) / _H
_LN_EPS = np.float32(1e-5)

_TB = 512


def _bases_concat(x):
    y = (x - _T0) * _INV_H
    d = [y - np.float32(j) for j in range(_NKNOT)]
    e = [np.float32(j) - y for j in range(_NKNOT)]
    zero = jnp.zeros_like(y)
    b = [jnp.maximum(jnp.minimum(d[i], e[i + 2]), zero) for i in range(10)]
    for k in (2, 3):
        b = [d[i] * b[i] - d[i + k + 1] * b[i + 1] for i in range(len(b) - 1)]
    return jnp.concatenate(b, axis=1)


def _mdot(a, b):
    return jax.lax.dot_general(a, b, (((1,), (0,)), ((), ())),
                               preferred_element_type=jnp.float32)


def _body(x_ref, bw0_ref, sw0_ref, bw1_ref, sw1_ref, g_ref, b_ref, o_ref):
    x = x_ref[...]
    h1 = _mdot(jax.nn.gelu(x), bw0_ref[...])
    h1 = h1 + _mdot(_bases_concat(x), sw0_ref[...])
    h2 = _mdot(jax.nn.gelu(h1), bw1_ref[...])
    h2 = h2 + _mdot(_bases_concat(h1), sw1_ref[...])
    mu = jnp.mean(h2, axis=-1, keepdims=True)
    xc = h2 - mu
    var = jnp.mean(xc * xc, axis=-1, keepdims=True)
    o_ref[...] = xc * jax.lax.rsqrt(var + _LN_EPS) * g_ref[...] + b_ref[...]


def kernel(x, base_w0, spline_w0, spline_s0, base_w1, spline_w1, spline_s1,
           ln_g, ln_b):
    B, D0 = x.shape
    D1 = base_w0.shape[0]
    D2 = base_w1.shape[0]

    bf16 = jnp.bfloat16
    bw0t = base_w0.T.astype(bf16)
    bw1t = base_w1.T.astype(bf16)
    sw0 = spline_w0 * (spline_s0 * np.float32(1.0 / 6.0))[..., None]
    sw0 = sw0.transpose(2, 1, 0).reshape(_GK * D0, D1).astype(bf16)
    sw1 = spline_w1 * (spline_s1 * np.float32(1.0 / 6.0))[..., None]
    sw1 = sw1.transpose(2, 1, 0).reshape(_GK * D1, D2).astype(bf16)
    g2 = ln_g.reshape(1, D2)
    b2 = ln_b.reshape(1, D2)

    grid = (B // _TB,)
    full = lambda i: (0, 0)
    out = pl.pallas_call(
        _body,
        grid=grid,
        in_specs=[
            pl.BlockSpec((_TB, D0), lambda i: (i, 0)),
            pl.BlockSpec((D0, D1), full),
            pl.BlockSpec((_GK * D0, D1), full),
            pl.BlockSpec((D1, D2), full),
            pl.BlockSpec((_GK * D1, D2), full),
            pl.BlockSpec((1, D2), full),
            pl.BlockSpec((1, D2), full),
        ],
        out_specs=pl.BlockSpec((_TB, D2), lambda i: (i, 0)),
        out_shape=jax.ShapeDtypeStruct((B, D2), jnp.float32),
        compiler_params=pltpu.CompilerParams(
            dimension_semantics=("parallel",),
        ),
    )(x, bw0t, sw0, bw1t, sw1, g2, b2)
    return out


# speedup vs baseline: 6.6748x; 1.0007x over previous
import jax
import jax.numpy as jnp
import numpy as np
from jax.experimental import pallas as pl
from jax.experimental.pallas import tpu as pltpu

_GRID_SIZE = 5
_ORDER = 3
_GK = _GRID_SIZE + _ORDER
_NKNOT = _GRID_SIZE + 2 * _ORDER + 1
_H = np.float32(2.0 / _GRID_SIZE)
_T0 = np.float32(-_ORDER) * _H + np.float32(-1.0)
_INV_H = np.float32(1.0) / _H
_LN_EPS = np.float32(1e-5)

_TB = 1024


def _bases_concat(x):
    y = (x - _T0) * _INV_H
    d = [y - np.float32(j) for j in range(_NKNOT)]
    e = [np.float32(j) - y for j in range(_NKNOT)]
    zero = jnp.zeros_like(y)
    b = [jnp.maximum(jnp.minimum(d[i], e[i + 2]), zero) for i in range(10)]
    for k in (2, 3):
        b = [d[i] * b[i] - d[i + k + 1] * b[i + 1] for i in range(len(b) - 1)]
    return jnp.concatenate(b, axis=1)


def _mdot(a, b):
    return jax.lax.dot_general(a, b, (((1,), (0,)), ((), ())),
                               preferred_element_type=jnp.float32)


def _body(x_ref, bw0_ref, sw0_ref, bw1_ref, sw1_ref, g_ref, b_ref, o_ref):
    x = x_ref[...]
    h1 = _mdot(jax.nn.gelu(x), bw0_ref[...])
    h1 = h1 + _mdot(_bases_concat(x), sw0_ref[...])
    h2 = _mdot(jax.nn.gelu(h1), bw1_ref[...])
    h2 = h2 + _mdot(_bases_concat(h1), sw1_ref[...])
    mu = jnp.mean(h2, axis=-1, keepdims=True)
    xc = h2 - mu
    var = jnp.mean(xc * xc, axis=-1, keepdims=True)
    o_ref[...] = xc * jax.lax.rsqrt(var + _LN_EPS) * g_ref[...] + b_ref[...]


def kernel(x, base_w0, spline_w0, spline_s0, base_w1, spline_w1, spline_s1,
           ln_g, ln_b):
    B, D0 = x.shape
    D1 = base_w0.shape[0]
    D2 = base_w1.shape[0]

    bf16 = jnp.bfloat16
    bw0t = base_w0.T.astype(bf16)
    bw1t = base_w1.T.astype(bf16)
    sw0 = spline_w0 * (spline_s0 * np.float32(1.0 / 6.0))[..., None]
    sw0 = sw0.transpose(2, 1, 0).reshape(_GK * D0, D1).astype(bf16)
    sw1 = spline_w1 * (spline_s1 * np.float32(1.0 / 6.0))[..., None]
    sw1 = sw1.transpose(2, 1, 0).reshape(_GK * D1, D2).astype(bf16)
    g2 = ln_g.reshape(1, D2)
    b2 = ln_b.reshape(1, D2)

    grid = (B // _TB,)
    full = lambda i: (0, 0)
    out = pl.pallas_call(
        _body,
        grid=grid,
        in_specs=[
            pl.BlockSpec((_TB, D0), lambda i: (i, 0)),
            pl.BlockSpec((D0, D1), full),
            pl.BlockSpec((_GK * D0, D1), full),
            pl.BlockSpec((D1, D2), full),
            pl.BlockSpec((_GK * D1, D2), full),
            pl.BlockSpec((1, D2), full),
            pl.BlockSpec((1, D2), full),
        ],
        out_specs=pl.BlockSpec((_TB, D2), lambda i: (i, 0)),
        out_shape=jax.ShapeDtypeStruct((B, D2), jnp.float32),
        compiler_params=pltpu.CompilerParams(
            dimension_semantics=("parallel",),
        ),
    )(x, bw0t, sw0, bw1t, sw1, g2, b2)
    return out


# speedup vs baseline: 8.2201x; 1.2315x over previous
import jax
import jax.numpy as jnp
import numpy as np
from jax.experimental import pallas as pl
from jax.experimental.pallas import tpu as pltpu

_GRID_SIZE = 5
_ORDER = 3
_GK = _GRID_SIZE + _ORDER
_NKNOT = _GRID_SIZE + 2 * _ORDER + 1
_H = np.float32(2.0 / _GRID_SIZE)
_T0 = np.float32(-_ORDER) * _H + np.float32(-1.0)
_INV_H = np.float32(1.0) / _H
_LN_EPS = np.float32(1e-5)

_TB = 512


def _bases_concat(x, dtype=jnp.float32):
    y = ((x - _T0) * _INV_H).astype(dtype)
    d = [y - dtype(j) for j in range(_NKNOT)]
    e = [dtype(j) - y for j in range(_NKNOT)]
    zero = jnp.zeros_like(y)
    b = [jnp.maximum(jnp.minimum(d[i], e[i + 2]), zero) for i in range(10)]
    for k in (2, 3):
        b = [d[i] * b[i] - d[i + k + 1] * b[i + 1] for i in range(len(b) - 1)]
    return jnp.concatenate(b, axis=1)


def _mdot(a, b):
    return jax.lax.dot_general(a, b, (((1,), (0,)), ((), ())),
                               preferred_element_type=jnp.float32)


def _body(x_ref, bw0_ref, sw0_ref, bw1_ref, sw1_ref, g_ref, b_ref, o_ref):
    x = x_ref[...]
    h1 = _mdot(jax.nn.gelu(x), bw0_ref[...])
    h1 = h1 + _mdot(_bases_concat(x), sw0_ref[...])
    h2 = _mdot(jax.nn.gelu(h1), bw1_ref[...])
    h2 = h2 + _mdot(_bases_concat(h1, jnp.bfloat16), sw1_ref[...])
    mu = jnp.mean(h2, axis=-1, keepdims=True)
    xc = h2 - mu
    var = jnp.mean(xc * xc, axis=-1, keepdims=True)
    o_ref[...] = xc * jax.lax.rsqrt(var + _LN_EPS) * g_ref[...] + b_ref[...]


def kernel(x, base_w0, spline_w0, spline_s0, base_w1, spline_w1, spline_s1,
           ln_g, ln_b):
    B, D0 = x.shape
    D1 = base_w0.shape[0]
    D2 = base_w1.shape[0]

    bf16 = jnp.bfloat16
    bw0t = base_w0.T.astype(bf16)
    bw1t = base_w1.T.astype(bf16)
    sw0 = spline_w0 * (spline_s0 * np.float32(1.0 / 6.0))[..., None]
    sw0 = sw0.transpose(2, 1, 0).reshape(_GK * D0, D1).astype(bf16)
    sw1 = spline_w1 * (spline_s1 * np.float32(1.0 / 6.0))[..., None]
    sw1 = sw1.transpose(2, 1, 0).reshape(_GK * D1, D2).astype(bf16)
    g2 = ln_g.reshape(1, D2)
    b2 = ln_b.reshape(1, D2)

    grid = (B // _TB,)
    full = lambda i: (0, 0)
    out = pl.pallas_call(
        _body,
        grid=grid,
        in_specs=[
            pl.BlockSpec((_TB, D0), lambda i: (i, 0)),
            pl.BlockSpec((D0, D1), full),
            pl.BlockSpec((_GK * D0, D1), full),
            pl.BlockSpec((D1, D2), full),
            pl.BlockSpec((_GK * D1, D2), full),
            pl.BlockSpec((1, D2), full),
            pl.BlockSpec((1, D2), full),
        ],
        out_specs=pl.BlockSpec((_TB, D2), lambda i: (i, 0)),
        out_shape=jax.ShapeDtypeStruct((B, D2), jnp.float32),
        compiler_params=pltpu.CompilerParams(
            dimension_semantics=("parallel",),
        ),
    )(x, bw0t, sw0, bw1t, sw1, g2, b2)
    return out
